# baseline (speedup 1.0000x reference)
import functools

import jax
import jax.numpy as jnp
from jax.experimental import pallas as pl
from jax.experimental.pallas import tpu as pltpu


def _round_up(x, m):
    return (x + m - 1) // m * m


def _linear_kernel(nchunk, ch, x_ref, w_hbm, b_ref, o_ref,
                   wbf_ref, land_ref, sem_ref):
    j = pl.program_id(0)
    i = pl.program_id(1)

    @pl.when(i == 0)
    def _fill():
        base = j * wbf_ref.shape[1]

        def start(c, slot):
            pltpu.make_async_copy(
                w_hbm.at[pl.ds(base + c * ch, ch), :],
                land_ref.at[slot],
                sem_ref.at[slot],
            ).start()

        start(0, 0)
        if nchunk > 1:
            start(1, 1)
        for c in range(nchunk):
            slot = c % 2
            pltpu.make_async_copy(
                land_ref.at[slot], land_ref.at[slot], sem_ref.at[slot]
            ).wait()
            wbf_ref[:, pl.ds(c * ch, ch)] = (
                land_ref[slot].astype(jnp.bfloat16).T)
            if c + 2 < nchunk:
                start(c + 2, slot)

    acc = jax.lax.dot_general(
        x_ref[...].astype(jnp.bfloat16), wbf_ref[...],
        dimension_numbers=(((1,), (0,)), ((), ())),
        preferred_element_type=jnp.float32,
    )
    o_ref[...] = acc + b_ref[...]


def kernel(x, weight, bias):
    M, K = x.shape
    N, K2 = weight.shape
    assert K == K2
    out_dtype = x.dtype

    bm = min(_round_up(M, 8), 512)
    bn = min(_round_up(N, 128), 2048)
    ch = min(bn, 256)
    Mp = _round_up(M, bm)
    Np = _round_up(N, bn)
    Kp = _round_up(K, 128)
    nchunk = bn // ch

    if (Mp, Kp) != (M, K):
        x = jnp.pad(x, ((0, Mp - M), (0, Kp - K)))
    if (Np, Kp) != (N, K):
        weight = jnp.pad(weight, ((0, Np - N), (0, Kp - K)))
    if Np != N:
        bias = jnp.pad(bias, (0, Np - N))
    b2d = bias.reshape(1, Np).astype(jnp.float32)

    cost = pl.CostEstimate(
        flops=2 * Mp * Np * Kp,
        transcendentals=0,
        bytes_accessed=4 * (Mp * Kp + Kp * Np + Mp * Np),
    )

    out = pl.pallas_call(
        functools.partial(_linear_kernel, nchunk, ch),
        out_shape=jax.ShapeDtypeStruct((Mp, Np), out_dtype),
        grid=(Np // bn, Mp // bm),
        in_specs=[
            pl.BlockSpec((bm, Kp), lambda j, i: (i, 0)),
            pl.BlockSpec(memory_space=pl.ANY),
            pl.BlockSpec((1, bn), lambda j, i: (0, j)),
        ],
        out_specs=pl.BlockSpec((bm, bn), lambda j, i: (i, j)),
        scratch_shapes=[
            pltpu.VMEM((Kp, bn), jnp.bfloat16),
            pltpu.VMEM((2, ch, Kp), jnp.float32),
            pltpu.SemaphoreType.DMA((2,)),
        ],
        compiler_params=pltpu.CompilerParams(
            dimension_semantics=("parallel", "arbitrary")
        ),
        cost_estimate=cost,
    )(x, weight, b2d)

    if (Mp, Np) != (M, N):
        out = out[:M, :N]
    return out


# speedup vs baseline: 1.0095x; 1.0095x over previous
import functools

import jax
import jax.numpy as jnp
from jax.experimental import pallas as pl
from jax.experimental.pallas import tpu as pltpu


def _round_up(x, m):
    return (x + m - 1) // m * m


def _linear_kernel(nchunk, ch, x_ref, w_hbm, b_ref, o_ref,
                   wbf_ref, land_ref, sem_ref):
    j = pl.program_id(0)
    i = pl.program_id(1)

    @pl.when(i == 0)
    def _fill_and_compute():
        base = j * wbf_ref.shape[1]

        def start(c, slot):
            pltpu.make_async_copy(
                w_hbm.at[pl.ds(base + c * ch, ch), :],
                land_ref.at[slot],
                sem_ref.at[slot],
            ).start()

        start(0, 0)
        if nchunk > 1:
            start(1, 1)
        xb = x_ref[...].astype(jnp.bfloat16)
        for c in range(nchunk):
            slot = c % 2
            pltpu.make_async_copy(
                land_ref.at[slot], land_ref.at[slot], sem_ref.at[slot]
            ).wait()
            sl = pl.ds(c * ch, ch)
            wbf_ref[:, sl] = land_ref[slot].astype(jnp.bfloat16).T
            if c + 2 < nchunk:
                start(c + 2, slot)
            acc_c = jax.lax.dot_general(
                xb, wbf_ref[:, sl],
                dimension_numbers=(((1,), (0,)), ((), ())),
                preferred_element_type=jnp.float32,
            )
            o_ref[:, sl] = acc_c + b_ref[:, sl]

    @pl.when(i > 0)
    def _compute():
        acc = jax.lax.dot_general(
            x_ref[...].astype(jnp.bfloat16), wbf_ref[...],
            dimension_numbers=(((1,), (0,)), ((), ())),
            preferred_element_type=jnp.float32,
        )
        o_ref[...] = acc + b_ref[...]


def kernel(x, weight, bias):
    M, K = x.shape
    N, K2 = weight.shape
    assert K == K2
    out_dtype = x.dtype

    bm = min(_round_up(M, 8), 512)
    bn = min(_round_up(N, 128), 2048)
    ch = min(bn, 256)
    Mp = _round_up(M, bm)
    Np = _round_up(N, bn)
    Kp = _round_up(K, 128)
    nchunk = bn // ch

    if (Mp, Kp) != (M, K):
        x = jnp.pad(x, ((0, Mp - M), (0, Kp - K)))
    if (Np, Kp) != (N, K):
        weight = jnp.pad(weight, ((0, Np - N), (0, Kp - K)))
    if Np != N:
        bias = jnp.pad(bias, (0, Np - N))
    b2d = bias.reshape(1, Np).astype(jnp.float32)

    cost = pl.CostEstimate(
        flops=2 * Mp * Np * Kp,
        transcendentals=0,
        bytes_accessed=4 * (Mp * Kp + Kp * Np + Mp * Np),
    )

    out = pl.pallas_call(
        functools.partial(_linear_kernel, nchunk, ch),
        out_shape=jax.ShapeDtypeStruct((Mp, Np), out_dtype),
        grid=(Np // bn, Mp // bm),
        in_specs=[
            pl.BlockSpec((bm, Kp), lambda j, i: (i, 0)),
            pl.BlockSpec(memory_space=pl.ANY),
            pl.BlockSpec((1, bn), lambda j, i: (0, j)),
        ],
        out_specs=pl.BlockSpec((bm, bn), lambda j, i: (i, j)),
        scratch_shapes=[
            pltpu.VMEM((Kp, bn), jnp.bfloat16),
            pltpu.VMEM((2, ch, Kp), jnp.float32),
            pltpu.SemaphoreType.DMA((2,)),
        ],
        compiler_params=pltpu.CompilerParams(
            dimension_semantics=("parallel", "arbitrary")
        ),
        cost_estimate=cost,
    )(x, weight, b2d)

    if (Mp, Np) != (M, N):
        out = out[:M, :N]
    return out


# speedup vs baseline: 1.0367x; 1.0270x over previous
import jax
import jax.numpy as jnp
from jax.experimental import pallas as pl
from jax.experimental.pallas import tpu as pltpu


def _round_up(x, m):
    return (x + m - 1) // m * m


def _linear_kernel(x_ref, w_ref, b_ref, o_ref):
    k = pl.program_id(2)

    @pl.when(k == 0)
    def _():
        o_ref[...] = jnp.broadcast_to(b_ref[...], o_ref.shape)

    o_ref[...] += jax.lax.dot_general(
        x_ref[...].astype(jnp.bfloat16),
        w_ref[...].astype(jnp.bfloat16),
        dimension_numbers=(((1,), (1,)), ((), ())),
        preferred_element_type=jnp.float32,
    )


def kernel(x, weight, bias):
    M, K = x.shape
    N, K2 = weight.shape
    assert K == K2
    out_dtype = x.dtype

    bm = min(_round_up(M, 8), 2048)
    bn = min(_round_up(N, 128), 2048)
    bk = min(_round_up(K, 128), 512)
    Mp = _round_up(M, bm)
    Np = _round_up(N, bn)
    Kp = _round_up(K, bk)

    if (Mp, Kp) != (M, K):
        x = jnp.pad(x, ((0, Mp - M), (0, Kp - K)))
    if (Np, Kp) != (N, K):
        weight = jnp.pad(weight, ((0, Np - N), (0, Kp - K)))
    if Np != N:
        bias = jnp.pad(bias, (0, Np - N))
    b2d = bias.reshape(1, Np).astype(jnp.float32)

    cost = pl.CostEstimate(
        flops=2 * Mp * Np * Kp,
        transcendentals=0,
        bytes_accessed=4 * (2 * Mp * Kp + 2 * Kp * Np + Mp * Np),
    )

    out = pl.pallas_call(
        _linear_kernel,
        out_shape=jax.ShapeDtypeStruct((Mp, Np), out_dtype),
        grid=(Mp // bm, Np // bn, Kp // bk),
        in_specs=[
            pl.BlockSpec((bm, bk), lambda i, j, k: (i, k)),
            pl.BlockSpec((bn, bk), lambda i, j, k: (j, k)),
            pl.BlockSpec((1, bn), lambda i, j, k: (0, j)),
        ],
        out_specs=pl.BlockSpec((bm, bn), lambda i, j, k: (i, j)),
        compiler_params=pltpu.CompilerParams(
            dimension_semantics=("parallel", "parallel", "arbitrary")
        ),
        cost_estimate=cost,
    )(x, weight, b2d)

    if (Mp, Np) != (M, N):
        out = out[:M, :N]
    return out
